# baseline (speedup 1.0000x reference)
import functools

import jax
import jax.numpy as jnp
from jax import lax
from jax.experimental import pallas as pl
from jax.experimental.pallas import tpu as pltpu
from jax.experimental.pallas import tpu_sc as plsc

VOCAB = 100000
EMBED_DIM = 300
BATCH = 1024
MAX_NORM = 1.0
V_TILE = 2048


@functools.lru_cache(maxsize=None)
def _sc_gather():
    info = plsc.get_sparse_core_info()
    nw = info.num_cores * info.num_subcores
    b_per_w = BATCH // nw
    mesh = plsc.VectorSubcoreMesh(core_axis_name="c", subcore_axis_name="s")

    @functools.partial(
        pl.kernel,
        mesh=mesh,
        out_type=jax.ShapeDtypeStruct((BATCH, EMBED_DIM), jnp.float32),
        scratch_types=[
            pltpu.VMEM((b_per_w,), jnp.int32),
            pltpu.VMEM((b_per_w, EMBED_DIM), jnp.float32),
            pltpu.SemaphoreType.DMA,
        ],
        compiler_params=pltpu.CompilerParams(needs_layout_passes=False),
    )
    def gather(table_hbm, idx_hbm, out_hbm, idx_v, rows_v, sem):
        wid = lax.axis_index("s") * info.num_cores + lax.axis_index("c")
        base = wid * b_per_w
        pltpu.sync_copy(idx_hbm.at[pl.ds(base, b_per_w)], idx_v)
        lanes = lax.iota(jnp.int32, 16)
        copies = []
        for i in range(b_per_w):
            vec = idx_v[pl.ds((i // 16) * 16, 16)]
            r = jnp.sum(jnp.where(lanes == (i % 16), vec, 0))
            c = pltpu.make_async_copy(
                table_hbm.at[pl.ds(r, 1)],
                rows_v.at[pl.ds(i, 1)],
                sem,
            )
            c.start()
            copies.append(c)
        for c in copies:
            c.wait()
        pltpu.sync_copy(rows_v, out_hbm.at[pl.ds(base, b_per_w)])

    return gather


def _proj_body(emb_ref, w_ref, b_ref, out_ref, esc_ref):
    @pl.when(pl.program_id(0) == 0)
    def _():
        e = emb_ref[...]
        nrm = jnp.sqrt(jnp.sum(e * e, axis=1, keepdims=True))
        scale = jnp.where(nrm > MAX_NORM, MAX_NORM / jnp.maximum(nrm, 1e-12), 1.0)
        esc_ref[...] = e * scale

    out_ref[...] = lax.dot_general(
        esc_ref[...],
        w_ref[...],
        dimension_numbers=(((1,), (1,)), ((), ())),
        preferred_element_type=jnp.float32,
    ) + b_ref[...]


def _tc_project(emb_raw, W, b2, interpret=False):
    grid = pl.cdiv(VOCAB, V_TILE)
    return pl.pallas_call(
        _proj_body,
        grid=(grid,),
        in_specs=[
            pl.BlockSpec((BATCH, EMBED_DIM), lambda j: (0, 0)),
            pl.BlockSpec((V_TILE, EMBED_DIM), lambda j: (j, 0)),
            pl.BlockSpec((1, V_TILE), lambda j: (0, j)),
        ],
        out_specs=pl.BlockSpec((BATCH, V_TILE), lambda j: (0, j)),
        out_shape=jax.ShapeDtypeStruct((BATCH, VOCAB), jnp.float32),
        scratch_shapes=[pltpu.VMEM((BATCH, EMBED_DIM), jnp.float32)],
        compiler_params=pltpu.CompilerParams(
            dimension_semantics=("arbitrary",)
        ),
        interpret=interpret,
    )(emb_raw, W, b2)


def kernel(inputs, emb_table, W, b):
    idx = inputs.reshape(-1).astype(jnp.int32)
    emb_raw = _sc_gather()(emb_table, idx)
    return _tc_project(emb_raw, W, b.reshape(1, VOCAB))
